# baseline (speedup 1.0000x reference)
import dataclasses
import functools

import jax
import jax.numpy as jnp
from jax import lax
from jax.experimental import pallas as pl
from jax.experimental.pallas import tpu as pltpu
from jax.experimental.pallas import tpu_sc as plsc


def _sc_compiler_params():
    cp = pltpu.CompilerParams()
    if "needs_layout_passes" in pltpu.CompilerParams.__dataclass_fields__:
        cp = dataclasses.replace(cp, needs_layout_passes=False)
    return cp


_C = 160
_NBUF = 2
_N_WORKERS = 32


def _sc_gather2(user_tab, item_tab, uidx, iidx):
    b = uidx.shape[0]
    d = user_tab.shape[1]
    per_w = b // _N_WORKERS
    nchunks = per_w // _C
    mesh = plsc.VectorSubcoreMesh(core_axis_name="c", subcore_axis_name="s")

    scratch = []
    for _ in range(_NBUF):
        scratch += [pltpu.VMEM((_C,), jnp.int32),
                    pltpu.VMEM((_C,), jnp.int32),
                    pltpu.VMEM((_C, d), user_tab.dtype),
                    pltpu.VMEM((_C, d), item_tab.dtype),
                    pltpu.VMEM((_C // 2, d), jnp.float32),
                    pltpu.VMEM((_C // 2, d), jnp.float32)]
    scratch += [pltpu.SemaphoreType.DMA] * (6 * _NBUF)

    @functools.partial(
        pl.kernel,
        out_type=[
            jax.ShapeDtypeStruct((b // 2, d), jnp.float32),
            jax.ShapeDtypeStruct((b // 2, d), jnp.float32),
        ],
        mesh=mesh,
        scratch_types=scratch,
        compiler_params=_sc_compiler_params(),
    )
    def gather_kernel(u_hbm, i_hbm, ui_hbm, ii_hbm, gu_hbm, gi_hbm, *scr):
        bufs = scr[:6 * _NBUF]
        sems = scr[6 * _NBUF:]

        def buf(bi, j):
            return bufs[6 * bi + j]

        def sem(bi, j):
            return sems[6 * bi + j]

        wid = lax.axis_index("s") * 2 + lax.axis_index("c")
        base = wid * per_w

        def idx_load(bi, off):
            return [
                pltpu.make_async_copy(ui_hbm.at[pl.ds(off, _C)], buf(bi, 0),
                                      sem(bi, 0)),
                pltpu.make_async_copy(ii_hbm.at[pl.ds(off, _C)], buf(bi, 1),
                                      sem(bi, 1)),
            ]

        def gath(bi):
            return [
                pltpu.make_async_copy(u_hbm.at[buf(bi, 0)], buf(bi, 2),
                                      sem(bi, 2)),
                pltpu.make_async_copy(i_hbm.at[buf(bi, 1)], buf(bi, 3),
                                      sem(bi, 3)),
            ]

        def wback(bi, row0):
            off2 = pl.multiple_of(row0 // 2, 8)
            return [
                pltpu.make_async_copy(buf(bi, 4),
                                      gu_hbm.at[pl.ds(off2, _C // 2)],
                                      sem(bi, 4)),
                pltpu.make_async_copy(buf(bi, 5),
                                      gi_hbm.at[pl.ds(off2, _C // 2)],
                                      sem(bi, 5)),
            ]

        def pack_rows(bi):
            for j in (2, 3):
                gb = buf(bi, j)
                pb = buf(bi, j + 2)

                @pl.loop(0, _C // 2)
                def _(r2):
                    for p in range(2):
                        for g in range(d // 32):
                            a = gb[2 * r2 + p, pl.ds(32 * g, 16)]
                            bvec = gb[2 * r2 + p, pl.ds(32 * g + 16, 16)]
                            pk = plsc.pack(a, bvec,
                                           format=plsc.PackFormat.INTERLEAVED)
                            pb[r2, pl.ds(64 * p + 16 * g, 16)] = (
                                plsc.bitcast(pk, jnp.float32))

        for bi in range(_NBUF):
            for c in idx_load(bi, base + bi * _C):
                c.start()

        @pl.loop(0, nchunks, step=_NBUF)
        def _(outer):
            for bi in range(_NBUF):
                @pl.when(outer >= _NBUF)
                def _():
                    for c in wback(bi, base):
                        c.wait()
                for c in idx_load(bi, base):
                    c.wait()
                for c in gath(bi):
                    c.start()
            for bi in range(_NBUF):
                g = outer + bi
                for c in gath(bi):
                    c.wait()
                pack_rows(bi)
                for c in wback(bi, base + g * _C):
                    c.start()

                @pl.when(outer + _NBUF < nchunks)
                def _():
                    for c in idx_load(bi, base + (g + _NBUF) * _C):
                        c.start()

        for bi in range(_NBUF):
            for c in wback(bi, base):
                c.wait()

    return gather_kernel(user_tab, item_tab, uidx, iidx)


def _unpack_bf16_pair(x_f32):
    bits = lax.bitcast_convert_type(x_f32, jnp.int32)
    lo = lax.bitcast_convert_type(bits << 16, jnp.float32)
    hi = lax.bitcast_convert_type(bits & jnp.int32(-65536), jnp.float32)
    return lo.astype(jnp.bfloat16), hi.astype(jnp.bfloat16)


def _tc_combine(review, gu, gi, wr, wu, wi):
    m, k = review.shape
    n = wr.shape[1]
    bm = 2000
    bm2 = bm // 2

    j_idx = jnp.arange(k // 2, dtype=jnp.int32)
    sig_a = 32 * (j_idx // 16) + (j_idx % 16)
    sig_b = sig_a + 16
    wu_a, wu_b = wu[sig_a], wu[sig_b]
    wi_a, wi_b = wi[sig_a], wi[sig_b]

    def body(r_ref, gu_ref, gi_ref, wr_ref, wua_ref, wub_ref, wia_ref,
             wib_ref, o_ref):
        r16 = r_ref[...].astype(jnp.bfloat16)
        racc = jnp.dot(r16, wr_ref[...].astype(jnp.bfloat16),
                       preferred_element_type=jnp.float32)

        def halves(g_ref, wa_ref, wb_ref):
            lo, hi = _unpack_bf16_pair(g_ref[...])
            wa16 = wa_ref[...].astype(jnp.bfloat16)
            wb16 = wb_ref[...].astype(jnp.bfloat16)
            ev = jnp.dot(lo[:, :k // 2], wa16,
                         preferred_element_type=jnp.float32)
            ev += jnp.dot(hi[:, :k // 2], wb16,
                          preferred_element_type=jnp.float32)
            od = jnp.dot(lo[:, k // 2:], wa16,
                         preferred_element_type=jnp.float32)
            od += jnp.dot(hi[:, k // 2:], wb16,
                          preferred_element_type=jnp.float32)
            return ev, od

        eu, ou = halves(gu_ref, wua_ref, wub_ref)
        ei, oi = halves(gi_ref, wia_ref, wib_ref)
        inter = jnp.stack([eu + ei, ou + oi], axis=1).reshape(bm, n)
        o_ref[...] = jnp.maximum(racc + inter, 0.0)

    row_spec = pl.BlockSpec((bm, k), lambda i: (i, 0))
    g_spec = pl.BlockSpec((bm2, k), lambda i: (i, 0))
    wr_spec = pl.BlockSpec((k, n), lambda i: (0, 0))
    wh_spec = pl.BlockSpec((k // 2, n), lambda i: (0, 0))
    return pl.pallas_call(
        body,
        grid=(m // bm,),
        in_specs=[row_spec, g_spec, g_spec, wr_spec,
                  wh_spec, wh_spec, wh_spec, wh_spec],
        out_specs=pl.BlockSpec((bm, n), lambda i: (i, 0)),
        out_shape=jax.ShapeDtypeStruct((m, n), jnp.float32),
    )(review, gu, gi, wr, wu_a, wu_b, wi_a, wi_b)


def kernel(review_feats, user_feats, item_feats, user_idx, item_idx, W):
    m, d = review_feats.shape

    pkey = jax.random.key(1)
    perm_i = jax.random.permutation(jax.random.fold_in(pkey, 0), d)
    perm_u = jax.random.permutation(jax.random.fold_in(pkey, 1), d)
    wr = W[0:d]
    wu = W[d:2 * d][jnp.argsort(perm_u)]
    wi = W[2 * d:3 * d][jnp.argsort(perm_i)]

    chunk = _C * _NBUF * _N_WORKERS
    bpad = ((m + chunk - 1) // chunk) * chunk
    pad_idx = jnp.arange(bpad - m, dtype=jnp.int32) % user_feats.shape[0]
    uidx = jnp.concatenate([user_idx.astype(jnp.int32), pad_idx])
    iidx = jnp.concatenate([item_idx.astype(jnp.int32), pad_idx])

    gu, gi = _sc_gather2(user_feats, item_feats, uidx, iidx)
    return _tc_combine(review_feats, gu, gi, wr, wu, wi)


# speedup vs baseline: 1.5592x; 1.5592x over previous
import dataclasses
import functools

import jax
import jax.numpy as jnp
from jax import lax
from jax.experimental import pallas as pl
from jax.experimental.pallas import tpu as pltpu
from jax.experimental.pallas import tpu_sc as plsc


def _sc_compiler_params():
    cp = pltpu.CompilerParams()
    if "needs_layout_passes" in pltpu.CompilerParams.__dataclass_fields__:
        cp = dataclasses.replace(cp, needs_layout_passes=False)
    return cp


_C = 160
_NBUF = 2
_N_WORKERS = 32


def _sc_gather2(user_tab, item_tab, uidx, iidx):
    b = uidx.shape[0]
    d = user_tab.shape[1]
    per_w = b // _N_WORKERS
    nchunks = per_w // _C
    mesh = plsc.VectorSubcoreMesh(core_axis_name="c", subcore_axis_name="s")

    scratch = []
    for _ in range(_NBUF):
        scratch += [pltpu.VMEM((_C,), jnp.int32),
                    pltpu.VMEM((_C,), jnp.int32),
                    pltpu.VMEM((_C, d), user_tab.dtype),
                    pltpu.VMEM((_C, d), item_tab.dtype),
                    pltpu.VMEM((_C // 2, d), jnp.float32),
                    pltpu.VMEM((_C // 2, d), jnp.float32)]
    scratch += [pltpu.SemaphoreType.DMA] * (6 * _NBUF)

    @functools.partial(
        pl.kernel,
        out_type=[
            jax.ShapeDtypeStruct((b // 2, d), jnp.float32),
            jax.ShapeDtypeStruct((b // 2, d), jnp.float32),
        ],
        mesh=mesh,
        scratch_types=scratch,
        compiler_params=_sc_compiler_params(),
    )
    def gather_kernel(u_hbm, i_hbm, ui_hbm, ii_hbm, gu_hbm, gi_hbm, *scr):
        bufs = scr[:6 * _NBUF]
        sems = scr[6 * _NBUF:]

        def buf(bi, j):
            return bufs[6 * bi + j]

        def sem(bi, j):
            return sems[6 * bi + j]

        wid = lax.axis_index("s") * 2 + lax.axis_index("c")
        base = wid * per_w

        def idx_load(bi, off):
            return [
                pltpu.make_async_copy(ui_hbm.at[pl.ds(off, _C)], buf(bi, 0),
                                      sem(bi, 0)),
                pltpu.make_async_copy(ii_hbm.at[pl.ds(off, _C)], buf(bi, 1),
                                      sem(bi, 1)),
            ]

        def gath(bi):
            return [
                pltpu.make_async_copy(u_hbm.at[buf(bi, 0)], buf(bi, 2),
                                      sem(bi, 2)),
                pltpu.make_async_copy(i_hbm.at[buf(bi, 1)], buf(bi, 3),
                                      sem(bi, 3)),
            ]

        def wback(bi, row0):
            off2 = pl.multiple_of(row0 // 2, 8)
            return [
                pltpu.make_async_copy(buf(bi, 4),
                                      gu_hbm.at[pl.ds(off2, _C // 2)],
                                      sem(bi, 4)),
                pltpu.make_async_copy(buf(bi, 5),
                                      gi_hbm.at[pl.ds(off2, _C // 2)],
                                      sem(bi, 5)),
            ]

        def pack_rows(bi):
            for j in (2, 3):
                gb = buf(bi, j)
                pb = buf(bi, j + 2)

                @functools.partial(plsc.parallel_loop, 0, _C // 2, unroll=4)
                def _(r2):
                    for p in range(2):
                        for g in range(d // 32):
                            a = gb[2 * r2 + p, pl.ds(32 * g, 16)]
                            bvec = gb[2 * r2 + p, pl.ds(32 * g + 16, 16)]
                            pk = plsc.pack(a, bvec,
                                           format=plsc.PackFormat.INTERLEAVED)
                            pb[r2, pl.ds(64 * p + 16 * g, 16)] = (
                                plsc.bitcast(pk, jnp.float32))

        for bi in range(_NBUF):
            for c in idx_load(bi, base + bi * _C):
                c.start()

        @pl.loop(0, nchunks, step=_NBUF)
        def _(outer):
            for bi in range(_NBUF):
                @pl.when(outer >= _NBUF)
                def _():
                    for c in wback(bi, base):
                        c.wait()
                for c in idx_load(bi, base):
                    c.wait()
                for c in gath(bi):
                    c.start()
            for bi in range(_NBUF):
                g = outer + bi
                for c in gath(bi):
                    c.wait()
                pack_rows(bi)
                for c in wback(bi, base + g * _C):
                    c.start()

                @pl.when(outer + _NBUF < nchunks)
                def _():
                    for c in idx_load(bi, base + (g + _NBUF) * _C):
                        c.start()

        for bi in range(_NBUF):
            for c in wback(bi, base):
                c.wait()

    return gather_kernel(user_tab, item_tab, uidx, iidx)


def _unpack_bf16_pair(x_f32):
    bits = lax.bitcast_convert_type(x_f32, jnp.int32)
    lo = lax.bitcast_convert_type(bits << 16, jnp.float32)
    hi = lax.bitcast_convert_type(bits & jnp.int32(-65536), jnp.float32)
    return lo.astype(jnp.bfloat16), hi.astype(jnp.bfloat16)


def _tc_combine(review, gu, gi, wr, wu, wi):
    m, k = review.shape
    n = wr.shape[1]
    bm = 2000
    bm2 = bm // 2

    j_idx = jnp.arange(k // 2, dtype=jnp.int32)
    sig_a = 32 * (j_idx // 16) + (j_idx % 16)
    sig_b = sig_a + 16
    wu_a, wu_b = wu[sig_a], wu[sig_b]
    wi_a, wi_b = wi[sig_a], wi[sig_b]

    def body(r_ref, gu_ref, gi_ref, wr_ref, wua_ref, wub_ref, wia_ref,
             wib_ref, o_ref):
        r16 = r_ref[...].astype(jnp.bfloat16)
        racc = jnp.dot(r16, wr_ref[...].astype(jnp.bfloat16),
                       preferred_element_type=jnp.float32)

        def halves(g_ref, wa_ref, wb_ref):
            lo, hi = _unpack_bf16_pair(g_ref[...])
            wa16 = wa_ref[...].astype(jnp.bfloat16)
            wb16 = wb_ref[...].astype(jnp.bfloat16)
            ev = jnp.dot(lo[:, :k // 2], wa16,
                         preferred_element_type=jnp.float32)
            ev += jnp.dot(hi[:, :k // 2], wb16,
                          preferred_element_type=jnp.float32)
            od = jnp.dot(lo[:, k // 2:], wa16,
                         preferred_element_type=jnp.float32)
            od += jnp.dot(hi[:, k // 2:], wb16,
                          preferred_element_type=jnp.float32)
            return ev, od

        eu, ou = halves(gu_ref, wua_ref, wub_ref)
        ei, oi = halves(gi_ref, wia_ref, wib_ref)
        inter = jnp.stack([eu + ei, ou + oi], axis=1).reshape(bm, n)
        o_ref[...] = jnp.maximum(racc + inter, 0.0)

    row_spec = pl.BlockSpec((bm, k), lambda i: (i, 0))
    g_spec = pl.BlockSpec((bm2, k), lambda i: (i, 0))
    wr_spec = pl.BlockSpec((k, n), lambda i: (0, 0))
    wh_spec = pl.BlockSpec((k // 2, n), lambda i: (0, 0))
    return pl.pallas_call(
        body,
        grid=(m // bm,),
        in_specs=[row_spec, g_spec, g_spec, wr_spec,
                  wh_spec, wh_spec, wh_spec, wh_spec],
        out_specs=pl.BlockSpec((bm, n), lambda i: (i, 0)),
        out_shape=jax.ShapeDtypeStruct((m, n), jnp.float32),
    )(review, gu, gi, wr, wu_a, wu_b, wi_a, wi_b)


def kernel(review_feats, user_feats, item_feats, user_idx, item_idx, W):
    m, d = review_feats.shape

    pkey = jax.random.key(1)
    perm_i = jax.random.permutation(jax.random.fold_in(pkey, 0), d)
    perm_u = jax.random.permutation(jax.random.fold_in(pkey, 1), d)
    wr = W[0:d]
    wu = W[d:2 * d][jnp.argsort(perm_u)]
    wi = W[2 * d:3 * d][jnp.argsort(perm_i)]

    chunk = _C * _NBUF * _N_WORKERS
    bpad = ((m + chunk - 1) // chunk) * chunk
    pad_idx = jnp.arange(bpad - m, dtype=jnp.int32) % user_feats.shape[0]
    uidx = jnp.concatenate([user_idx.astype(jnp.int32), pad_idx])
    iidx = jnp.concatenate([item_idx.astype(jnp.int32), pad_idx])

    gu, gi = _sc_gather2(user_feats, item_feats, uidx, iidx)
    return _tc_combine(review_feats, gu, gi, wr, wu, wi)
